# baseline (speedup 1.0000x reference)
import functools

import jax
import jax.numpy as jnp
from jax import lax
from jax.experimental import pallas as pl
from jax.experimental.pallas import tpu as pltpu
from jax.experimental.pallas import tpu_sc as plsc

N = 10000
E = 320000
D = 128
HEADS1 = 5
OUT = 128
G = 64

NPAD = 10368
BR = 1296
EP = 330240
SCAN_CH = 2064
SCAN_ROUNDS = EP // SCAN_CH
SCAN_IT = SCAN_CH // 16
LISTCAP = 2688

PASSES1 = 4
ROWS1 = 80
K1 = 16
PASSES2 = 2
ROWS2 = 160
K2 = 32
NCORES = 2
NSUB = 16


def _lin2(din, dout):
  nb = NPAD // BR

  def body(x_ref, w1_ref, b1_ref, w2_ref, b2_ref, o_ref):
    i = pl.program_id(0)
    xv = x_ref[...]

    @pl.when(i < nb)
    def _():
      o_ref[...] = jnp.dot(xv, w1_ref[...], preferred_element_type=jnp.float32) + b1_ref[...]

    @pl.when(i >= nb)
    def _():
      o_ref[...] = jnp.dot(xv, w2_ref[...], preferred_element_type=jnp.float32) + b2_ref[...]

  return pl.pallas_call(
      body,
      grid=(2 * nb,),
      in_specs=[
          pl.BlockSpec((BR, din), lambda i: (i % nb, 0)),
          pl.BlockSpec((din, dout), lambda i: (0, 0)),
          pl.BlockSpec((1, dout), lambda i: (0, 0)),
          pl.BlockSpec((din, dout), lambda i: (0, 0)),
          pl.BlockSpec((1, dout), lambda i: (0, 0)),
      ],
      out_specs=pl.BlockSpec((BR, dout), lambda i: (i, 0)),
      out_shape=jax.ShapeDtypeStruct((2 * NPAD, dout), jnp.float32),
  )


def _norm_lin2(heads, dh, dout):
  din = heads * dh

  nb = NPAD // BR

  def body(a_ref, s_ref, b_ref, w1_ref, c1_ref, w2_ref, c2_ref, o_ref):
    i = pl.program_id(0)
    parts = []
    for h in range(heads):
      num = a_ref[:, h * dh:(h + 1) * dh]
      den = s_ref[:, h:h + 1] + 1e-16
      parts.append(num / den)
    v = (jnp.concatenate(parts, axis=1) if heads > 1 else parts[0]) + b_ref[...]
    x1 = jnp.maximum(v, 0.1 * v)

    @pl.when(i < nb)
    def _():
      o_ref[...] = jnp.dot(x1, w1_ref[...], preferred_element_type=jnp.float32) + c1_ref[...]

    @pl.when(i >= nb)
    def _():
      o_ref[...] = jnp.dot(x1, w2_ref[...], preferred_element_type=jnp.float32) + c2_ref[...]

  return pl.pallas_call(
      body,
      grid=(2 * nb,),
      in_specs=[
          pl.BlockSpec((BR, din), lambda i: (i % nb, 0)),
          pl.BlockSpec((BR, 16), lambda i: (i % nb, 0)),
          pl.BlockSpec((1, din), lambda i: (0, 0)),
          pl.BlockSpec((din, dout), lambda i: (0, 0)),
          pl.BlockSpec((1, dout), lambda i: (0, 0)),
          pl.BlockSpec((din, dout), lambda i: (0, 0)),
          pl.BlockSpec((1, dout), lambda i: (0, 0)),
      ],
      out_specs=pl.BlockSpec((BR, dout), lambda i: (i, 0)),
      out_shape=jax.ShapeDtypeStruct((2 * NPAD, dout), jnp.float32),
  )


def _pool_fc():
  grid = NPAD // BR

  def body(a_ref, s_ref, b_ref, oh_ref, w_ref, c_ref, o_ref):
    i = pl.program_id(0)

    @pl.when(i == 0)
    def _():
      o_ref[...] = jnp.full((G, OUT), -jnp.inf, jnp.float32)

    v = a_ref[...] / (s_ref[:, 0:1] + 1e-16) + b_ref[...]
    x2 = jnp.maximum(v, 0.1 * v)
    oh = oh_ref[...]
    rows = []
    for g in range(G):
      sel = jnp.where(oh[:, g:g + 1] > 0.0, x2, -jnp.inf)
      rows.append(jnp.max(sel, axis=0, keepdims=True))
    local = jnp.concatenate(rows, axis=0)
    o_ref[...] = jnp.maximum(o_ref[...], local)

    @pl.when(i == grid - 1)
    def _():
      p = o_ref[...]
      p = jnp.where(p == -jnp.inf, 0.0, p)
      r = jnp.dot(p, w_ref[...], preferred_element_type=jnp.float32) + c_ref[...]
      o_ref[...] = jnp.maximum(r, 0.1 * r)

  return pl.pallas_call(
      body,
      grid=(grid,),
      in_specs=[
          pl.BlockSpec((BR, OUT), lambda i: (i, 0)),
          pl.BlockSpec((BR, 16), lambda i: (i, 0)),
          pl.BlockSpec((1, OUT), lambda i: (0, 0)),
          pl.BlockSpec((BR, G), lambda i: (i, 0)),
          pl.BlockSpec((OUT, OUT), lambda i: (0, 0)),
          pl.BlockSpec((1, OUT), lambda i: (0, 0)),
      ],
      out_specs=pl.BlockSpec((G, OUT), lambda i: (0, 0)),
      out_shape=jax.ShapeDtypeStruct((G, OUT), jnp.float32),
  )


def _edge_pass(heads, dh, passes, rows_pt, kch):
  row = heads * dh
  rows_acc = rows_pt + 1
  span = NCORES * NSUB * rows_pt
  ngroup = kch // 16
  mesh = plsc.VectorSubcoreMesh(
      core_axis_name="c", subcore_axis_name="s",
      num_cores=NCORES, num_subcores=NSUB)

  @functools.partial(
      pl.kernel,
      out_type=[
          jax.ShapeDtypeStruct((NPAD, row), jnp.float32),
          jax.ShapeDtypeStruct((NPAD, 16), jnp.float32),
      ],
      mesh=mesh,
      compiler_params=pltpu.CompilerParams(needs_layout_passes=False),
      scratch_types=[
          pltpu.VMEM((rows_acc, row), jnp.float32),
          pltpu.VMEM((rows_acc, 16), jnp.float32),
          pltpu.VMEM((4, SCAN_CH), jnp.int32),
          pltpu.VMEM((LISTCAP,), jnp.int32),
          pltpu.VMEM((LISTCAP,), jnp.int32),
          pltpu.VMEM((2 * kch, row), jnp.float32),
          pltpu.VMEM((2 * kch, row), jnp.float32),
          pltpu.VMEM((kch, 16), jnp.float32),
          pltpu.VMEM((row,), jnp.float32),
          pltpu.VMEM((2 * kch,), jnp.int32),
          pltpu.VMEM((2 * kch,), jnp.int32),
          pltpu.SemaphoreType.DMA,
          pltpu.SemaphoreType.DMA,
          pltpu.SemaphoreType.DMA,
      ],
  )
  def k(xlr_h, sd_h, att_h, outa_h, outs_h,
        accum, acc_s, sbuf, lst_s, lst_d,
        xlr_a, xlr_b, ast, att_v, cidx_a, cidx_b, sem1, semb, sem2):
    core = lax.axis_index("c")
    sub = lax.axis_index("s")
    wid = core * NSUB + sub
    i16 = lax.iota(jnp.int32, 16)
    zero16 = jnp.zeros((16,), jnp.float32)

    pltpu.sync_copy(att_h, att_v)

    def stage_chunk(pos, base, dummy, cidx, xlr_v, sem):
      relg = []
      for g in range(ngroup):
        d16 = lst_d[pl.ds(pos + g * 16, 16)]
        s16 = lst_s[pl.ds(pos + g * 16, 16)]
        if dummy is not None:
          d16 = jnp.where(dummy, jnp.full((16,), rows_pt, jnp.int32) + base,
                          d16)
          s16 = jnp.where(dummy, jnp.zeros((16,), jnp.int32), s16)
        cidx[pl.ds(kch + g * 16, 16)] = jnp.minimum(d16, NPAD - 1) + NPAD
        cidx[pl.ds(g * 16, 16)] = s16
        relg.append(d16 - base)
      return pltpu.async_copy(xlr_h.at[cidx], xlr_v, sem), relg

    def compute_chunk(relg, base, xlr_v):
      for g in range(ngroup):
        rows = i16 + g * 16

        @plsc.parallel_loop(0, dh, unroll=4, carry=(zero16,) * heads)
        def accs(c, acc):
          new = []
          for h in range(heads):
            col = jnp.full((16,), h * dh, jnp.int32) + c
            a = plsc.load_gather(xlr_v, [rows, col])
            b = plsc.load_gather(xlr_v, [rows + kch, col])
            z = a + b
            z = jnp.maximum(z, 0.2 * z)
            new.append(acc[h] + z * plsc.load_gather(att_v, [col]))
          return tuple(new)

        avs = [jnp.exp(a) for a in accs]
        for h in range(heads):
          plsc.store_scatter(ast, [rows, jnp.full((16,), h, jnp.int32)],
                             avs[h])
        if heads < 16:
          for h in range(heads, 16):
            plsc.store_scatter(ast, [rows, jnp.full((16,), h, jnp.int32)],
                               zero16)

        rels = [relg[g][e] for e in range(16)]
        aes = [[avs[h][e] for e in range(16)] for h in range(heads)]
        for e in range(16):
          plsc.addupdate(acc_s.at[rels[e]], ast[g * 16 + e])
        for h in range(heads):

          @plsc.parallel_loop(0, dh // 16, unroll=2)
          def _(cc):
            c0 = h * dh + cc * 16
            for e in range(16):
              v = xlr_v[g * 16 + e, pl.ds(c0, 16)] * aes[h][e]
              plsc.addupdate(accum.at[rels[e], pl.ds(c0, 16)], v)

    @pl.loop(0, passes, init_carry=jnp.int32(0))
    def _(p, cp):
      base = p * span + wid * rows_pt

      @plsc.parallel_loop(0, rows_acc)
      def _(r):
        for cc in range(row // 16):
          accum[r, pl.ds(cc * 16, 16)] = zero16
        acc_s[r, pl.ds(0, 16)] = zero16

      pltpu.async_copy(sd_h.at[pl.ds(0, 2)], sbuf.at[pl.ds(0, 2)], sem2)

      @pl.loop(0, SCAN_ROUNDS, init_carry=jnp.int32(0))
      def m(rd, off):
        slot = (rd % 2) * 2
        nslot = 2 - slot
        pltpu.make_async_copy(sd_h.at[pl.ds(rd * 2, 2)],
                              sbuf.at[pl.ds(slot, 2)], sem2).wait()

        @pl.when(rd + 1 < SCAN_ROUNDS)
        def _():
          pltpu.async_copy(sd_h.at[pl.ds((rd + 1) * 2, 2)],
                           sbuf.at[pl.ds(nslot, 2)], sem2)

        @pl.loop(0, SCAN_IT, init_carry=off)
        def off(j, o):
          d = sbuf[slot, pl.ds(j * 16, 16)]
          s = sbuf[slot + 1, pl.ds(j * 16, 16)]
          rel = d - base
          msk = (rel >= 0) & (rel < rows_pt)
          plsc.store_compressed(lst_d.at[pl.ds(o, 16)], d, mask=msk)
          plsc.store_compressed(lst_s.at[pl.ds(o, 16)], s, mask=msk)
          cnt = jnp.max(plsc.all_reduce_population_count(msk))
          return o + cnt

        last = rd == SCAN_ROUNDS - 1

        @pl.when(last)
        def _():
          for t in range(ngroup):
            lst_d[pl.ds(off + t * 16, 16)] = (
                jnp.full((16,), rows_pt, jnp.int32) + base)
            lst_s[pl.ds(off + t * 16, 16)] = jnp.zeros((16,), jnp.int32)

        q = jnp.where(last, (off + kch) // kch, off // kch)

        @pl.loop(0, (q + 1) // 2, init_carry=jnp.int32(0))
        def _(j, c):
          b_dummy = 2 * j + 1 >= q
          cpa, rga = stage_chunk(2 * j * kch, base, None, cidx_a, xlr_a,
                                 sem1)
          cpb, rgb = stage_chunk((2 * j + 1) * kch, base, b_dummy, cidx_b,
                                 xlr_b, semb)
          cpa.wait()
          compute_chunk(rga, base, xlr_a)
          cpb.wait()
          compute_chunk(rgb, base, xlr_b)
          return c

        for t in range(ngroup):
          vd = lst_d[pl.ds(q * kch + t * 16, 16)]
          vs = lst_s[pl.ds(q * kch + t * 16, 16)]
          lst_d[pl.ds(t * 16, 16)] = vd
          lst_s[pl.ds(t * 16, 16)] = vs
        return jnp.maximum(off - q * kch, 0)

      @pl.when(base + rows_pt <= NPAD)
      def _():
        pltpu.sync_copy(accum.at[pl.ds(0, rows_pt)],
                        outa_h.at[pl.ds(base, rows_pt)])
        pltpu.sync_copy(acc_s.at[pl.ds(0, rows_pt)],
                        outs_h.at[pl.ds(base, rows_pt)])
      return cp

  return k


def kernel(x, edge_index, batch, Wl1, bl1, Wr1, br1, att1, b1,
           Wl2, bl2, Wr2, br2, att2, b2, Wfc, bfc):
  f32 = jnp.float32
  i32 = jnp.int32
  loops = jnp.arange(N, dtype=i32)
  srcp = jnp.concatenate(
      [edge_index[0].astype(i32), loops, jnp.zeros((EP - E - N,), i32)])
  dstp = jnp.concatenate(
      [edge_index[1].astype(i32), loops,
       jnp.full((EP - E - N,), 1 << 30, i32)])
  xpad = jnp.pad(x, ((0, NPAD - N), (0, 0)))

  xlr1 = _lin2(D, HEADS1 * D)(
      xpad, Wl1, bl1.reshape(1, -1), Wr1, br1.reshape(1, -1))

  sd = jnp.stack([dstp.reshape(SCAN_ROUNDS, SCAN_CH),
                  srcp.reshape(SCAN_ROUNDS, SCAN_CH)],
                 axis=1).reshape(2 * SCAN_ROUNDS, SCAN_CH)
  ua, us = _edge_pass(HEADS1, D, PASSES1, ROWS1, K1)(
      xlr1, sd, att1.reshape(-1))

  xlr2 = _norm_lin2(HEADS1, D, OUT)(
      ua, us, b1.reshape(1, -1), Wl2, bl2.reshape(1, -1),
      Wr2, br2.reshape(1, -1))

  va, vs = _edge_pass(1, OUT, PASSES2, ROWS2, K2)(
      xlr2, sd, att2.reshape(-1))

  batch_p = jnp.concatenate([batch.astype(i32), jnp.full((NPAD - N,), G, i32)])
  oh = (batch_p[:, None] == jnp.arange(G, dtype=i32)[None, :]).astype(f32)
  out = _pool_fc()(va, vs, b2.reshape(1, -1), oh, Wfc, bfc.reshape(1, -1))
  return out


# speedup vs baseline: 2.2442x; 2.2442x over previous
import functools

import jax
import jax.numpy as jnp
from jax import lax
from jax.experimental import pallas as pl
from jax.experimental.pallas import tpu as pltpu
from jax.experimental.pallas import tpu_sc as plsc

N = 10000
E = 320000
D = 128
HEADS1 = 5
OUT = 128
G = 64

NPAD = 10368
BR = 1296
EP = 330240
SCAN_CH = 2064
SCAN_ROUNDS = EP // SCAN_CH
SCAN_IT = SCAN_CH // 16
LISTCAP = 2688

PASSES1 = 4
ROWS1 = 80
K1 = 32
PASSES2 = 2
ROWS2 = 160
K2 = 64
NCORES = 2
NSUB = 16


def _lin2(din, dout):
  nb = NPAD // BR

  def body(x_ref, w1_ref, b1_ref, w2_ref, b2_ref, o_ref):
    i = pl.program_id(0)
    xv = x_ref[...]

    @pl.when(i < nb)
    def _():
      o_ref[...] = jnp.dot(xv, w1_ref[...], preferred_element_type=jnp.float32) + b1_ref[...]

    @pl.when(i >= nb)
    def _():
      o_ref[...] = jnp.dot(xv, w2_ref[...], preferred_element_type=jnp.float32) + b2_ref[...]

  return pl.pallas_call(
      body,
      grid=(2 * nb,),
      in_specs=[
          pl.BlockSpec((BR, din), lambda i: (i % nb, 0)),
          pl.BlockSpec((din, dout), lambda i: (0, 0)),
          pl.BlockSpec((1, dout), lambda i: (0, 0)),
          pl.BlockSpec((din, dout), lambda i: (0, 0)),
          pl.BlockSpec((1, dout), lambda i: (0, 0)),
      ],
      out_specs=pl.BlockSpec((BR, dout), lambda i: (i, 0)),
      out_shape=jax.ShapeDtypeStruct((2 * NPAD, dout), jnp.float32),
  )


def _norm_lin2(heads, dh, dout):
  din = heads * dh

  nb = NPAD // BR

  def body(a_ref, s_ref, b_ref, w1_ref, c1_ref, w2_ref, c2_ref, o_ref):
    i = pl.program_id(0)
    parts = []
    for h in range(heads):
      num = a_ref[:, h * dh:(h + 1) * dh]
      den = s_ref[:, h:h + 1] + 1e-16
      parts.append(num / den)
    v = (jnp.concatenate(parts, axis=1) if heads > 1 else parts[0]) + b_ref[...]
    x1 = jnp.maximum(v, 0.1 * v)

    @pl.when(i < nb)
    def _():
      o_ref[...] = jnp.dot(x1, w1_ref[...], preferred_element_type=jnp.float32) + c1_ref[...]

    @pl.when(i >= nb)
    def _():
      o_ref[...] = jnp.dot(x1, w2_ref[...], preferred_element_type=jnp.float32) + c2_ref[...]

  return pl.pallas_call(
      body,
      grid=(2 * nb,),
      in_specs=[
          pl.BlockSpec((BR, din), lambda i: (i % nb, 0)),
          pl.BlockSpec((BR, 16), lambda i: (i % nb, 0)),
          pl.BlockSpec((1, din), lambda i: (0, 0)),
          pl.BlockSpec((din, dout), lambda i: (0, 0)),
          pl.BlockSpec((1, dout), lambda i: (0, 0)),
          pl.BlockSpec((din, dout), lambda i: (0, 0)),
          pl.BlockSpec((1, dout), lambda i: (0, 0)),
      ],
      out_specs=pl.BlockSpec((BR, dout), lambda i: (i, 0)),
      out_shape=jax.ShapeDtypeStruct((2 * NPAD, dout), jnp.float32),
  )


def _pool_fc():
  grid = NPAD // BR

  def body(a_ref, s_ref, b_ref, oh_ref, w_ref, c_ref, o_ref):
    i = pl.program_id(0)

    @pl.when(i == 0)
    def _():
      o_ref[...] = jnp.full((G, OUT), -jnp.inf, jnp.float32)

    v = a_ref[...] / (s_ref[:, 0:1] + 1e-16) + b_ref[...]
    x2 = jnp.maximum(v, 0.1 * v)
    oh = oh_ref[...]
    rows = []
    for g in range(G):
      sel = jnp.where(oh[:, g:g + 1] > 0.0, x2, -jnp.inf)
      rows.append(jnp.max(sel, axis=0, keepdims=True))
    local = jnp.concatenate(rows, axis=0)
    o_ref[...] = jnp.maximum(o_ref[...], local)

    @pl.when(i == grid - 1)
    def _():
      p = o_ref[...]
      p = jnp.where(p == -jnp.inf, 0.0, p)
      r = jnp.dot(p, w_ref[...], preferred_element_type=jnp.float32) + c_ref[...]
      o_ref[...] = jnp.maximum(r, 0.1 * r)

  return pl.pallas_call(
      body,
      grid=(grid,),
      in_specs=[
          pl.BlockSpec((BR, OUT), lambda i: (i, 0)),
          pl.BlockSpec((BR, 16), lambda i: (i, 0)),
          pl.BlockSpec((1, OUT), lambda i: (0, 0)),
          pl.BlockSpec((BR, G), lambda i: (i, 0)),
          pl.BlockSpec((OUT, OUT), lambda i: (0, 0)),
          pl.BlockSpec((1, OUT), lambda i: (0, 0)),
      ],
      out_specs=pl.BlockSpec((G, OUT), lambda i: (0, 0)),
      out_shape=jax.ShapeDtypeStruct((G, OUT), jnp.float32),
  )


def _edge_pass(heads, dh, passes, rows_pt, kch):
  row = heads * dh
  rows_acc = rows_pt + 1
  span = NCORES * NSUB * rows_pt
  ngroup = kch // 16
  mesh = plsc.VectorSubcoreMesh(
      core_axis_name="c", subcore_axis_name="s",
      num_cores=NCORES, num_subcores=NSUB)

  @functools.partial(
      pl.kernel,
      out_type=[
          jax.ShapeDtypeStruct((NPAD, row), jnp.float32),
          jax.ShapeDtypeStruct((NPAD, 16), jnp.float32),
      ],
      mesh=mesh,
      compiler_params=pltpu.CompilerParams(needs_layout_passes=False,
                                           use_tc_tiling_on_sc=False),
      scratch_types=[
          pltpu.VMEM((rows_acc, row), jnp.float32),
          pltpu.VMEM((rows_acc, 16), jnp.float32),
          pltpu.VMEM((4, SCAN_CH), jnp.int32),
          pltpu.VMEM((LISTCAP,), jnp.int32),
          pltpu.VMEM((LISTCAP,), jnp.int32),
          pltpu.VMEM((2 * kch, row), jnp.float32),
          pltpu.VMEM((kch, 16), jnp.float32),
          pltpu.VMEM((row,), jnp.float32),
          pltpu.VMEM((2 * kch,), jnp.int32),
          pltpu.SemaphoreType.DMA,
          pltpu.SemaphoreType.DMA,
      ],
  )
  def k(xlr_h, sd_h, att_h, outa_h, outs_h,
        accum, acc_s, sbuf, lst_s, lst_d,
        xlr_v, ast, att_v, cidx, sem1, sem2):
    core = lax.axis_index("c")
    sub = lax.axis_index("s")
    wid = core * NSUB + sub
    i16 = lax.iota(jnp.int32, 16)
    zero16 = jnp.zeros((16,), jnp.float32)

    pltpu.sync_copy(att_h, att_v)

    def do_chunk(pos, base):
      relg = []
      for g in range(ngroup):
        d16 = lst_d[pl.ds(pos + g * 16, 16)]
        s16 = lst_s[pl.ds(pos + g * 16, 16)]
        cidx[pl.ds(kch + g * 16, 16)] = jnp.minimum(d16, NPAD - 1) + NPAD
        cidx[pl.ds(g * 16, 16)] = s16
        relg.append(d16 - base)
      pltpu.async_copy(xlr_h.at[cidx], xlr_v, sem1).wait()
      for g in range(ngroup):
        rows = i16 + g * 16

        @plsc.parallel_loop(0, dh, unroll=4, carry=(zero16,) * heads)
        def accs(c, acc):
          new = []
          for h in range(heads):
            col = jnp.full((16,), h * dh, jnp.int32) + c
            a = plsc.load_gather(xlr_v, [rows, col])
            b = plsc.load_gather(xlr_v, [rows + kch, col])
            z = a + b
            z = jnp.maximum(z, 0.2 * z)
            new.append(acc[h] + z * plsc.load_gather(att_v, [col]))
          return tuple(new)

        avs = [jnp.exp(a) for a in accs]
        for h in range(heads):
          plsc.store_scatter(ast, [rows, jnp.full((16,), h, jnp.int32)],
                             avs[h])
        if heads < 16:
          for h in range(heads, 16):
            plsc.store_scatter(ast, [rows, jnp.full((16,), h, jnp.int32)],
                               zero16)

        rels = [relg[g][e] for e in range(16)]
        aes = [[avs[h][e] for e in range(16)] for h in range(heads)]
        for e in range(16):
          plsc.addupdate(acc_s.at[rels[e]], ast[g * 16 + e])
        for h in range(heads):

          @plsc.parallel_loop(0, dh // 16, unroll=2)
          def _(cc):
            c0 = h * dh + cc * 16
            for e in range(16):
              v = xlr_v[g * 16 + e, pl.ds(c0, 16)] * aes[h][e]
              plsc.addupdate(accum.at[rels[e], pl.ds(c0, 16)], v)

    @pl.loop(0, passes, init_carry=jnp.int32(0))
    def _(p, cp):
      base = p * span + wid * rows_pt

      @plsc.parallel_loop(0, rows_acc)
      def _(r):
        for cc in range(row // 16):
          accum[r, pl.ds(cc * 16, 16)] = zero16
        acc_s[r, pl.ds(0, 16)] = zero16

      pltpu.async_copy(sd_h.at[pl.ds(0, 2)], sbuf.at[pl.ds(0, 2)], sem2)

      @pl.loop(0, SCAN_ROUNDS, init_carry=jnp.int32(0))
      def m(rd, off):
        slot = (rd % 2) * 2
        nslot = 2 - slot
        pltpu.make_async_copy(sd_h.at[pl.ds(rd * 2, 2)],
                              sbuf.at[pl.ds(slot, 2)], sem2).wait()

        @pl.when(rd + 1 < SCAN_ROUNDS)
        def _():
          pltpu.async_copy(sd_h.at[pl.ds((rd + 1) * 2, 2)],
                           sbuf.at[pl.ds(nslot, 2)], sem2)

        @pl.loop(0, SCAN_IT, init_carry=off)
        def off(j, o):
          d = sbuf[slot, pl.ds(j * 16, 16)]
          s = sbuf[slot + 1, pl.ds(j * 16, 16)]
          rel = d - base
          msk = (rel >= 0) & (rel < rows_pt)
          plsc.store_compressed(lst_d.at[pl.ds(o, 16)], d, mask=msk)
          plsc.store_compressed(lst_s.at[pl.ds(o, 16)], s, mask=msk)
          cnt = jnp.max(plsc.all_reduce_population_count(msk))
          return o + cnt

        last = rd == SCAN_ROUNDS - 1

        @pl.when(last)
        def _():
          for t in range(ngroup):
            lst_d[pl.ds(off + t * 16, 16)] = (
                jnp.full((16,), rows_pt, jnp.int32) + base)
            lst_s[pl.ds(off + t * 16, 16)] = jnp.zeros((16,), jnp.int32)

        q = jnp.where(last, (off + kch) // kch, off // kch)

        @pl.loop(0, q, init_carry=jnp.int32(0))
        def _(j, c):
          do_chunk(j * kch, base)
          return c

        for t in range(ngroup):
          vd = lst_d[pl.ds(q * kch + t * 16, 16)]
          vs = lst_s[pl.ds(q * kch + t * 16, 16)]
          lst_d[pl.ds(t * 16, 16)] = vd
          lst_s[pl.ds(t * 16, 16)] = vs
        return jnp.maximum(off - q * kch, 0)

      @pl.when(base + rows_pt <= NPAD)
      def _():
        pltpu.sync_copy(accum.at[pl.ds(0, rows_pt)],
                        outa_h.at[pl.ds(base, rows_pt)])
        pltpu.sync_copy(acc_s.at[pl.ds(0, rows_pt)],
                        outs_h.at[pl.ds(base, rows_pt)])
      return cp

  return k


def kernel(x, edge_index, batch, Wl1, bl1, Wr1, br1, att1, b1,
           Wl2, bl2, Wr2, br2, att2, b2, Wfc, bfc):
  f32 = jnp.float32
  i32 = jnp.int32
  loops = jnp.arange(N, dtype=i32)
  srcp = jnp.concatenate(
      [edge_index[0].astype(i32), loops, jnp.zeros((EP - E - N,), i32)])
  dstp = jnp.concatenate(
      [edge_index[1].astype(i32), loops,
       jnp.full((EP - E - N,), 1 << 30, i32)])
  xpad = jnp.pad(x, ((0, NPAD - N), (0, 0)))

  xlr1 = _lin2(D, HEADS1 * D)(
      xpad, Wl1, bl1.reshape(1, -1), Wr1, br1.reshape(1, -1))

  sd = jnp.stack([dstp.reshape(SCAN_ROUNDS, SCAN_CH),
                  srcp.reshape(SCAN_ROUNDS, SCAN_CH)],
                 axis=1).reshape(2 * SCAN_ROUNDS, SCAN_CH)
  ua, us = _edge_pass(HEADS1, D, PASSES1, ROWS1, K1)(
      xlr1, sd, att1.reshape(-1))

  xlr2 = _norm_lin2(HEADS1, D, OUT)(
      ua, us, b1.reshape(1, -1), Wl2, bl2.reshape(1, -1),
      Wr2, br2.reshape(1, -1))

  va, vs = _edge_pass(1, OUT, PASSES2, ROWS2, K2)(
      xlr2, sd, att2.reshape(-1))

  batch_p = jnp.concatenate([batch.astype(i32), jnp.full((NPAD - N,), G, i32)])
  oh = (batch_p[:, None] == jnp.arange(G, dtype=i32)[None, :]).astype(f32)
  out = _pool_fc()(va, vs, b2.reshape(1, -1), oh, Wfc, bfc.reshape(1, -1))
  return out
